# baseline (speedup 1.0000x reference)
import functools

import jax
import jax.numpy as jnp
from jax import lax
from jax.experimental import pallas as pl
from jax.experimental.pallas import tpu as pltpu
from jax.experimental.pallas import tpu_sc as plsc

EMBED = 1024
T_LEN = 2048
BATCH = 4
NUM_CORES = 2
NUM_SUBCORES = 16
NW = NUM_CORES * NUM_SUBCORES
T_PER_W = T_LEN // NW
CHUNK = 16
NCHUNK = T_PER_W // CHUNK
NSTEP = NCHUNK * BATCH
NBUF = 3
LANES = 16
VPR = EMBED // LANES


def _build_kernel():
  mesh = plsc.VectorSubcoreMesh(core_axis_name="c", subcore_axis_name="s")

  @functools.partial(
      pl.kernel,
      mesh=mesh,
      out_type=jax.ShapeDtypeStruct((BATCH * T_LEN, EMBED), jnp.float32),
      scratch_types=[
          pltpu.VMEM((NSTEP * CHUNK,), jnp.int32),
          pltpu.VMEM((T_PER_W, EMBED), jnp.float32),
          pltpu.VMEM((CHUNK, EMBED), jnp.float32),
          pltpu.VMEM((CHUNK, EMBED), jnp.float32),
          pltpu.VMEM((CHUNK, EMBED), jnp.float32),
          pltpu.SemaphoreType.DMA,
          pltpu.SemaphoreType.DMA,
          pltpu.SemaphoreType.DMA,
          pltpu.SemaphoreType.DMA,
          pltpu.SemaphoreType.DMA,
          pltpu.SemaphoreType.DMA,
          pltpu.SemaphoreType.DMA,
          pltpu.SemaphoreType.DMA,
      ],
  )
  def k(idx_hbm, tok_hbm, pos_hbm, out_hbm, idx_v, pos_v, tok0_v, tok1_v,
        tok2_v, sem_i, sem_p, sem_g0, sem_g1, sem_g2, sem_s0, sem_s1, sem_s2):
    wid = lax.axis_index("s") * NUM_CORES + lax.axis_index("c")
    t_base = wid * T_PER_W

    def step_row0(k_):
      b, tc = divmod(k_, NCHUNK)
      return b * T_LEN + t_base + tc * CHUNK

    pos_copy = pltpu.async_copy(
        pos_hbm.at[pl.ds(t_base, T_PER_W)], pos_v, sem_p)

    for b in range(BATCH):
      pltpu.async_copy(
          idx_hbm.at[pl.ds(b * T_LEN + t_base, T_PER_W)],
          idx_v.at[pl.ds(b * T_PER_W, T_PER_W)], sem_i)
    pltpu.make_async_copy(
        idx_hbm.at[pl.ds(0, NSTEP * CHUNK)], idx_v, sem_i).wait()

    bufs = (tok0_v, tok1_v, tok2_v)
    gather_sems = (sem_g0, sem_g1, sem_g2)
    store_sems = (sem_s0, sem_s1, sem_s2)

    def start_gather(k_):
      p = k_ % NBUF
      return pltpu.async_copy(
          tok_hbm.at[idx_v.at[pl.ds(k_ * CHUNK, CHUNK)]],
          bufs[p], gather_sems[p])

    gathers = [None] * NSTEP
    stores = [None] * NSTEP
    gathers[0] = start_gather(0)
    for k_ in range(NSTEP):
      p = k_ % NBUF
      if k_ + 1 < NSTEP:
        if k_ >= 2:
          stores[k_ - 2].wait()
        gathers[k_ + 1] = start_gather(k_ + 1)
      gathers[k_].wait()
      if k_ == 0:
        pos_copy.wait()

      buf = bufs[p]
      prow = (k_ % NCHUNK) * CHUNK

      def add_row(r, _):
        for jj in range(VPR):
          d = jj * LANES
          plsc.addupdate(
              buf.at[r, pl.ds(d, LANES)], pos_v[prow + r, pl.ds(d, LANES)])
        return 0

      lax.fori_loop(0, CHUNK, add_row, 0)
      stores[k_] = pltpu.async_copy(
          buf, out_hbm.at[pl.ds(step_row0(k_), CHUNK)], store_sems[p])
    stores[NSTEP - 2].wait()
    stores[NSTEP - 1].wait()

  return k


_kernel = _build_kernel()


def kernel(idx, token_table, pos_table):
  b, t = idx.shape
  idx_flat = jnp.reshape(idx.astype(jnp.int32), (b * t,))
  out = _kernel(idx_flat, token_table, pos_table)
  return jnp.reshape(out, (b, t, token_table.shape[1]))


# speedup vs baseline: 1.0727x; 1.0727x over previous
import functools

import jax
import jax.numpy as jnp
from jax import lax
from jax.experimental import pallas as pl
from jax.experimental.pallas import tpu as pltpu
from jax.experimental.pallas import tpu_sc as plsc

EMBED = 1024
T_LEN = 2048
BATCH = 4
NUM_CORES = 2
NUM_SUBCORES = 16
NW = NUM_CORES * NUM_SUBCORES
T_PER_W = T_LEN // NW
CHUNK = 32
NCHUNK = T_PER_W // CHUNK
NSTEP = NCHUNK * BATCH
LANES = 16
VPR = EMBED // LANES


def _build_kernel():
  mesh = plsc.VectorSubcoreMesh(core_axis_name="c", subcore_axis_name="s")

  @functools.partial(
      pl.kernel,
      mesh=mesh,
      out_type=jax.ShapeDtypeStruct((BATCH * T_LEN, EMBED), jnp.float32),
      scratch_types=[
          pltpu.VMEM((BATCH * T_PER_W,), jnp.int32),
          pltpu.VMEM((CHUNK, EMBED), jnp.float32),
          pltpu.VMEM((CHUNK, EMBED), jnp.float32),
          pltpu.VMEM((CHUNK, EMBED), jnp.float32),
          pltpu.SemaphoreType.DMA,
          pltpu.SemaphoreType.DMA,
          pltpu.SemaphoreType.DMA,
          pltpu.SemaphoreType.DMA,
          pltpu.SemaphoreType.DMA,
          pltpu.SemaphoreType.DMA,
      ],
  )
  def k(idx_hbm, tok_hbm, pos_hbm, out_hbm, idx_v, tok0_v, tok1_v, pos_v,
        sem_i, sem_p, sem_g0, sem_g1, sem_s0, sem_s1):
    wid = lax.axis_index("s") * NUM_CORES + lax.axis_index("c")
    t_base = wid * T_PER_W

    def step_row0(k_):
      tc = k_ // BATCH
      b = k_ % BATCH
      return b * T_LEN + t_base + tc * CHUNK

    def idx_off(k_):
      tc = k_ // BATCH
      b = k_ % BATCH
      return b * T_PER_W + tc * CHUNK

    def idx_copy(b):
      return pltpu.async_copy(
          idx_hbm.at[pl.ds(b * T_LEN + t_base, T_PER_W)],
          idx_v.at[pl.ds(b * T_PER_W, T_PER_W)], sem_i)

    bufs = (tok0_v, tok1_v)
    gather_sems = (sem_g0, sem_g1)
    store_sems = (sem_s0, sem_s1)

    def start_gather(k_, p):
      return pltpu.async_copy(
          tok_hbm.at[idx_v.at[pl.ds(idx_off(k_), CHUNK)]],
          bufs[p], gather_sems[p])

    def pos_copy(tc):
      return pltpu.async_copy(
          pos_hbm.at[pl.ds(t_base + tc * CHUNK, CHUNK)], pos_v, sem_p)

    idx_copy(0).wait()
    start_gather(0, 0)
    pos_copy(0)
    for b in range(1, BATCH):
      idx_copy(b)
    pltpu.make_async_copy(
        idx_hbm.at[pl.ds(0, (BATCH - 1) * T_PER_W)],
        idx_v.at[pl.ds(T_PER_W, (BATCH - 1) * T_PER_W)], sem_i).wait()

    def pair_body(j, _):
      for p in range(2):
        k_ = 2 * j + p
        buf = bufs[p]

        @pl.when(k_ >= 1)
        def _():
          pltpu.make_async_copy(
              bufs[1 - p], out_hbm.at[pl.ds(step_row0(k_ - 1), CHUNK)],
              store_sems[1 - p]).wait()

        @pl.when(k_ + 1 < NSTEP)
        def _():
          start_gather(k_ + 1, 1 - p)

        pltpu.make_async_copy(
            tok_hbm.at[idx_v.at[pl.ds(idx_off(k_), CHUNK)]], buf,
            gather_sems[p]).wait()

        @pl.when(k_ % BATCH == 0)
        def _():
          pltpu.make_async_copy(
              pos_hbm.at[pl.ds(t_base, CHUNK)], pos_v, sem_p).wait()

        def add_row(r, _):
          for jj in range(VPR):
            d = jj * LANES
            plsc.addupdate(
                buf.at[r, pl.ds(d, LANES)], pos_v[r, pl.ds(d, LANES)])
          return 0

        lax.fori_loop(0, CHUNK, add_row, 0)
        pltpu.async_copy(
            buf, out_hbm.at[pl.ds(step_row0(k_), CHUNK)], store_sems[p])

        @pl.when(k_ == BATCH - 1)
        def _():
          pos_copy(1)
      return 0

    lax.fori_loop(0, NSTEP // 2, pair_body, 0)
    pltpu.make_async_copy(
        bufs[1], out_hbm.at[pl.ds(step_row0(NSTEP - 1), CHUNK)],
        store_sems[1]).wait()

  return k


_kernel = _build_kernel()


def kernel(idx, token_table, pos_table):
  b, t = idx.shape
  idx_flat = jnp.reshape(idx.astype(jnp.int32), (b * t,))
  out = _kernel(idx_flat, token_table, pos_table)
  return jnp.reshape(out, (b, t, token_table.shape[1]))
